# baseline (speedup 1.0000x reference)
import functools

import jax
import jax.numpy as jnp
from jax import lax
from jax.experimental import pallas as pl
from jax.experimental.pallas import tpu as pltpu
from jax.experimental.pallas import tpu_sc as plsc

_NC = 2
_NS = 16
_L = 16
_C = 32


def _pe_add_body(x_hbm, pe_hbm, out_hbm, pe_v, x_v):
    B = out_hbm.shape[0]
    S = out_hbm.shape[1]
    D = out_hbm.shape[2]
    nw = _NC * _NS
    sw = S // nw
    wid = lax.axis_index("s") * _NC + lax.axis_index("c")
    s_base = wid * sw
    jpr = D // _L

    def chunk_body(kk, _):
        s0 = s_base + kk * _C
        pltpu.sync_copy(pe_hbm.at[pl.ds(s0, _C)], pe_v)

        def b_body(b, _):
            pltpu.sync_copy(x_hbm.at[b, pl.ds(s0, _C)], x_v)

            def add_body(i, _):
                r = i // jpr
                off = (i % jpr) * _L
                x_v[r, pl.ds(off, _L)] = (
                    x_v[r, pl.ds(off, _L)] + pe_v[r, pl.ds(off, _L)]
                )
                return 0

            lax.fori_loop(0, _C * jpr, add_body, 0)
            pltpu.sync_copy(x_v, out_hbm.at[b, pl.ds(s0, _C)])
            return 0

        lax.fori_loop(0, B, b_body, 0)
        return 0

    lax.fori_loop(0, sw // _C, chunk_body, 0)


def kernel(x, pe):
    B, S, D = x.shape
    mesh = plsc.VectorSubcoreMesh(core_axis_name="c", subcore_axis_name="s")
    run = functools.partial(
        pl.kernel,
        mesh=mesh,
        out_type=jax.ShapeDtypeStruct((B, S, D), x.dtype),
        scratch_types=[
            pltpu.VMEM((_C, D), jnp.float32),
            pltpu.VMEM((_C, D), jnp.float32),
        ],
    )(_pe_add_body)
    return run(x, pe)


# speedup vs baseline: 3.0506x; 3.0506x over previous
import functools

import jax
import jax.numpy as jnp
from jax import lax
from jax.experimental import pallas as pl
from jax.experimental.pallas import tpu as pltpu
from jax.experimental.pallas import tpu_sc as plsc

_NC = 2
_NS = 16
_L = 16
_C = 4
_NBUF = 4
_JB = 16


def _pe_add_body(x_hbm, pe_hbm, out_hbm, xb, peb,
                 ls0, ls1, ls2, ls3, ss0, ss1, ss2, ss3):
    B = out_hbm.shape[0]
    S = out_hbm.shape[1]
    D = out_hbm.shape[2]
    nw = _NC * _NS
    sw = S // nw
    n_steps = sw // _C
    n_outer = n_steps // _NBUF
    wid = lax.axis_index("s") * _NC + lax.axis_index("c")
    s_base = wid * sw
    lsems = [ls0, ls1, ls2, ls3]
    ssems = [ss0, ss1, ss2, ss3]

    def issue_loads(step, u):
        s0 = s_base + step * _C
        pltpu.async_copy(x_hbm.at[:, pl.ds(s0, _C)], xb.at[u], lsems[u])
        pltpu.async_copy(pe_hbm.at[pl.ds(s0, _C)], peb.at[u], lsems[u])

    def wait_loads(u):
        pltpu.make_async_copy(x_hbm.at[:, pl.ds(0, _C)], xb.at[u],
                              lsems[u]).wait()
        pltpu.make_async_copy(pe_hbm.at[pl.ds(0, _C)], peb.at[u],
                              lsems[u]).wait()

    def issue_store(step, u):
        s0 = s_base + step * _C
        pltpu.async_copy(xb.at[u], out_hbm.at[:, pl.ds(s0, _C)], ssems[u])

    def wait_store(u):
        pltpu.make_async_copy(xb.at[u], out_hbm.at[:, pl.ds(0, _C)],
                              ssems[u]).wait()

    def compute(u):
        def row_body(r, _):
            for jb in range(D // (_JB * _L)):
                base = jb * _JB * _L
                pe_vs = [peb[u, r, pl.ds(base + i * _L, _L)]
                         for i in range(_JB)]
                for b in range(B):
                    for i in range(_JB):
                        off = base + i * _L
                        xb[u, b, r, pl.ds(off, _L)] = (
                            xb[u, b, r, pl.ds(off, _L)] + pe_vs[i]
                        )
            return 0

        lax.fori_loop(0, _C, row_body, 0)

    issue_loads(0, 0)
    issue_loads(1, 1)

    def outer(kk4, _):
        for u in range(_NBUF):
            kk = kk4 * _NBUF + u
            pu = (u + 2) % _NBUF
            if u < 2:
                @pl.when(kk4 > 0)
                def _():
                    wait_store(pu)
                issue_loads(kk + 2, pu)
            else:
                wait_store(pu)

                @pl.when(kk4 < n_outer - 1)
                def _():
                    issue_loads(kk + 2, pu)
            wait_loads(u)
            compute(u)
            issue_store(kk, u)
        return 0

    lax.fori_loop(0, n_outer, outer, 0)
    wait_store(2)
    wait_store(3)


def kernel(x, pe):
    B, S, D = x.shape
    mesh = plsc.VectorSubcoreMesh(core_axis_name="c", subcore_axis_name="s")
    run = functools.partial(
        pl.kernel,
        mesh=mesh,
        out_type=jax.ShapeDtypeStruct((B, S, D), x.dtype),
        scratch_types=[
            pltpu.VMEM((_NBUF, B, _C, D), jnp.float32),
            pltpu.VMEM((_NBUF, _C, D), jnp.float32),
            pltpu.SemaphoreType.DMA,
            pltpu.SemaphoreType.DMA,
            pltpu.SemaphoreType.DMA,
            pltpu.SemaphoreType.DMA,
            pltpu.SemaphoreType.DMA,
            pltpu.SemaphoreType.DMA,
            pltpu.SemaphoreType.DMA,
            pltpu.SemaphoreType.DMA,
        ],
    )(_pe_add_body)
    return run(x, pe)
